# baseline (speedup 1.0000x reference)
import functools

import jax
import jax.numpy as jnp
from jax import lax
from jax.experimental import pallas as pl
from jax.experimental.pallas import tpu as pltpu
from jax.experimental.pallas import tpu_sc as plsc

B = 16384
D = 64
L = 16
NC = 2
NS = 16
NW = NC * NS
BPW = B // NW
CH = 128
NSB = BPW // CH

_mesh = plsc.VectorSubcoreMesh(core_axis_name="c", subcore_axis_name="s")


@functools.partial(
    pl.kernel,
    mesh=_mesh,
    compiler_params=pltpu.CompilerParams(needs_layout_passes=False),
    out_type=jax.ShapeDtypeStruct((B,), jnp.float32),
    scratch_types=[
        pltpu.VMEM((NSB, CH), jnp.int32),
        pltpu.VMEM((NSB, CH), jnp.int32),
        pltpu.VMEM((NSB, CH), jnp.int32),
        pltpu.VMEM((NSB, CH), jnp.int32),
        pltpu.VMEM((CH, 2 * D), jnp.float32),
        pltpu.VMEM((CH, 2 * D), jnp.float32),
        pltpu.VMEM((CH, 2 * D), jnp.float32),
        pltpu.VMEM((CH, 2 * D), jnp.float32),
        pltpu.VMEM((BPW,), jnp.float32),
        pltpu.SemaphoreType.DMA,
        pltpu.SemaphoreType.DMA,
    ],
)
def _mf_sc(user_hbm, item_hbm, utab_hbm, itab_hbm, out_hbm,
           uidx, iidx, uhalf, ihalf, ub0, ub1, ib0, ib1, oacc, sem0, sem1):
    wid = lax.axis_index("s") * NC + lax.axis_index("c")
    base = wid * BPW
    ubufs, ibufs, sems = (ub0, ub1), (ib0, ib1), (sem0, sem1)

    for j in range(NSB):
        pltpu.sync_copy(user_hbm.at[pl.ds(base + j * CH, CH)], uidx.at[j])
        pltpu.sync_copy(item_hbm.at[pl.ds(base + j * CH, CH)], iidx.at[j])
    for j in range(NSB):
        for g in range(CH // L):
            sl = pl.ds(g * L, L)
            uhalf[j, sl] = lax.shift_right_logical(uidx[j, sl], 1)
            ihalf[j, sl] = lax.shift_right_logical(iidx[j, sl], 1)

    def fire(s):
        p = s % 2
        return (
            pltpu.async_copy(utab_hbm.at[uhalf.at[s]], ubufs[p], sems[p]),
            pltpu.async_copy(itab_hbm.at[ihalf.at[s]], ibufs[p], sems[p]),
        )

    lane = lax.broadcasted_iota(jnp.int32, (L,), 0)

    inflight = fire(0)
    for s in range(NSB):
        for c in inflight:
            c.wait()
        if s + 1 < NSB:
            nxt = fire(s + 1)
        ub, ib = ubufs[s % 2], ibufs[s % 2]

        def group_body(g, carry, ub=ub, ib=ib, s=s):
            uvec = uidx[s, pl.ds(g * L, L)]
            ivec = iidx[s, pl.ds(g * L, L)]
            acc = jnp.zeros((L,), jnp.float32)
            for k in range(L):
                uoff = (uvec[k] & 1) * D
                ioff = (ivec[k] & 1) * D
                b = g * L + k
                p = jnp.zeros((L,), jnp.float32)
                for f in range(0, D, L):
                    u = ub[b, pl.ds(uoff + f, L)]
                    v = ib[b, pl.ds(ioff + f, L)]
                    p = p + u * v
                acc = jnp.where(lane == k, jnp.sum(p), acc)
            oacc[pl.ds(s * CH + g * L, L)] = acc
            return carry

        lax.fori_loop(0, CH // L, group_body, 0)
        if s + 1 < NSB:
            inflight = nxt

    pltpu.sync_copy(oacc, out_hbm.at[pl.ds(base, BPW)])


def kernel(user, item, user_factors, item_factors):
    utab = user_factors.reshape(50000, 2 * D)
    itab = item_factors.reshape(50000, 2 * D)
    return _mf_sc(user.astype(jnp.int32), item.astype(jnp.int32), utab, itab)


# speedup vs baseline: 1.4246x; 1.4246x over previous
import functools

import jax
import jax.numpy as jnp
from jax import lax
from jax.experimental import pallas as pl
from jax.experimental.pallas import tpu as pltpu
from jax.experimental.pallas import tpu_sc as plsc

B = 16384
D = 64
L = 16
NC = 2
NS = 16
NW = NC * NS
BPW = B // NW
CH = 128
NSB = BPW // CH

_mesh = plsc.VectorSubcoreMesh(core_axis_name="c", subcore_axis_name="s")


@functools.partial(
    pl.kernel,
    mesh=_mesh,
    compiler_params=pltpu.CompilerParams(needs_layout_passes=False),
    out_type=jax.ShapeDtypeStruct((B,), jnp.float32),
    scratch_types=[
        pltpu.VMEM((NSB, CH), jnp.int32),
        pltpu.VMEM((NSB, CH), jnp.int32),
        pltpu.VMEM((CH, D), jnp.float32),
        pltpu.VMEM((CH, D), jnp.float32),
        pltpu.VMEM((CH, D), jnp.float32),
        pltpu.VMEM((CH, D), jnp.float32),
        pltpu.VMEM((BPW,), jnp.float32),
        pltpu.SemaphoreType.DMA,
        pltpu.SemaphoreType.DMA,
        pltpu.SemaphoreType.DMA,
        pltpu.SemaphoreType.DMA,
    ],
)
def _mf_sc(user_hbm, item_hbm, utab_hbm, itab_hbm, out_hbm,
           uidx, iidx, ub0, ub1, ib0, ib1, oacc, us0, us1, is0, is1):
    wid = lax.axis_index("s") * NC + lax.axis_index("c")
    base = wid * BPW
    ubufs, ibufs = (ub0, ub1), (ib0, ib1)
    usems, isems = (us0, us1), (is0, is1)

    for j in range(NSB):
        pltpu.sync_copy(user_hbm.at[pl.ds(base + j * CH, CH)], uidx.at[j])
        pltpu.sync_copy(item_hbm.at[pl.ds(base + j * CH, CH)], iidx.at[j])

    def fire(s):
        p = s % 2

        def g_body(g, carry):
            uvec = uidx[s, pl.ds(g * L, L)]
            ivec = iidx[s, pl.ds(g * L, L)]
            for k in range(L):
                m = g * L + k
                pltpu.async_copy(utab_hbm.at[pl.ds(uvec[k], 1), :],
                                 ubufs[p].at[pl.ds(m, 1), :], usems[p])
                pltpu.async_copy(itab_hbm.at[pl.ds(ivec[k], 1), :],
                                 ibufs[p].at[pl.ds(m, 1), :], isems[p])
            return carry

        lax.fori_loop(0, CH // L, g_body, 0)

    def drain(s):
        p = s % 2
        pltpu.make_async_copy(utab_hbm.at[pl.ds(0, CH), :], ubufs[p],
                              usems[p]).wait()
        pltpu.make_async_copy(itab_hbm.at[pl.ds(0, CH), :], ibufs[p],
                              isems[p]).wait()

    lane = lax.broadcasted_iota(jnp.int32, (L,), 0)

    fire(0)
    for s in range(NSB):
        drain(s)
        if s + 1 < NSB:
            fire(s + 1)
        ub, ib = ubufs[s % 2], ibufs[s % 2]

        def group_body(g, carry, ub=ub, ib=ib, s=s):
            acc = jnp.zeros((L,), jnp.float32)
            for k in range(L):
                b = g * L + k
                p = jnp.zeros((L,), jnp.float32)
                for f in range(0, D, L):
                    u = ub[b, pl.ds(f, L)]
                    v = ib[b, pl.ds(f, L)]
                    p = p + u * v
                acc = jnp.where(lane == k, jnp.sum(p), acc)
            oacc[pl.ds(s * CH + g * L, L)] = acc
            return carry

        lax.fori_loop(0, CH // L, group_body, 0)

    pltpu.sync_copy(oacc, out_hbm.at[pl.ds(base, BPW)])


def kernel(user, item, user_factors, item_factors):
    return _mf_sc(user.astype(jnp.int32), item.astype(jnp.int32),
                  user_factors, item_factors)


# speedup vs baseline: 1.7503x; 1.2286x over previous
import functools

import jax
import jax.numpy as jnp
from jax import lax
from jax.experimental import pallas as pl
from jax.experimental.pallas import tpu as pltpu
from jax.experimental.pallas import tpu_sc as plsc

B = 16384
D = 64
L = 16
NC = 2
NS = 16
NW = NC * NS
BPW = B // NW
CH = 128
NSB = BPW // CH

_mesh = plsc.VectorSubcoreMesh(core_axis_name="c", subcore_axis_name="s")


@functools.partial(
    pl.kernel,
    mesh=_mesh,
    compiler_params=pltpu.CompilerParams(needs_layout_passes=False),
    out_type=jax.ShapeDtypeStruct((B,), jnp.float32),
    scratch_types=[
        pltpu.VMEM((NSB, CH), jnp.int32),
        pltpu.VMEM((NSB, CH), jnp.int32),
        pltpu.VMEM((CH // 8, 8, D), jnp.float32),
        pltpu.VMEM((CH // 8, 8, D), jnp.float32),
        pltpu.VMEM((CH // 8, 8, D), jnp.float32),
        pltpu.VMEM((CH // 8, 8, D), jnp.float32),
        pltpu.VMEM((BPW,), jnp.float32),
        pltpu.SemaphoreType.DMA,
        pltpu.SemaphoreType.DMA,
        pltpu.SemaphoreType.DMA,
        pltpu.SemaphoreType.DMA,
    ],
)
def _mf_sc(user_hbm, item_hbm, utab_hbm, itab_hbm, out_hbm,
           uidx, iidx, ub0, ub1, ib0, ib1, oacc, us0, us1, is0, is1):
    wid = lax.axis_index("s") * NC + lax.axis_index("c")
    base = wid * BPW
    ubufs, ibufs = (ub0, ub1), (ib0, ib1)
    usems, isems = (us0, us1), (is0, is1)

    for j in range(NSB):
        pltpu.sync_copy(user_hbm.at[pl.ds(base + j * CH, CH)], uidx.at[j])
        pltpu.sync_copy(item_hbm.at[pl.ds(base + j * CH, CH)], iidx.at[j])

    def fire(s):
        p = s % 2

        def g_body(g, carry):
            uvec = uidx[s, pl.ds(g * L, L)]
            ivec = iidx[s, pl.ds(g * L, L)]
            for k in range(L):
                m = g * L + k
                ur, ir = uvec[k], ivec[k]
                pltpu.async_copy(
                    utab_hbm.at[pl.ds(ur >> 3, 1), ur & 7, :],
                    ubufs[p].at[pl.ds(m // 8, 1), m % 8, :], usems[p])
                pltpu.async_copy(
                    itab_hbm.at[pl.ds(ir >> 3, 1), ir & 7, :],
                    ibufs[p].at[pl.ds(m // 8, 1), m % 8, :], isems[p])
            return carry

        lax.fori_loop(0, CH // L, g_body, 0)

    def drain(s):
        p = s % 2
        pltpu.make_async_copy(utab_hbm.at[pl.ds(0, CH // 8), :, :], ubufs[p],
                              usems[p]).wait()
        pltpu.make_async_copy(itab_hbm.at[pl.ds(0, CH // 8), :, :], ibufs[p],
                              isems[p]).wait()

    lane = lax.broadcasted_iota(jnp.int32, (L,), 0)

    fire(0)
    for s in range(NSB):
        drain(s)
        if s + 1 < NSB:
            fire(s + 1)
        ub, ib = ubufs[s % 2], ibufs[s % 2]

        def group_body(g, carry, ub=ub, ib=ib, s=s):
            acc = jnp.zeros((L,), jnp.float32)
            for k in range(L):
                b = g * L + k
                p = jnp.zeros((L,), jnp.float32)
                for f in range(0, D, L):
                    u = ub[b // 8, b % 8, pl.ds(f, L)]
                    v = ib[b // 8, b % 8, pl.ds(f, L)]
                    p = p + u * v
                acc = jnp.where(lane == k, jnp.sum(p), acc)
            oacc[pl.ds(s * CH + g * L, L)] = acc
            return carry

        lax.fori_loop(0, CH // L, group_body, 0)

    pltpu.sync_copy(oacc, out_hbm.at[pl.ds(base, BPW)])


def kernel(user, item, user_factors, item_factors):
    utab = user_factors.reshape(12500, 8, D)
    itab = item_factors.reshape(12500, 8, D)
    return _mf_sc(user.astype(jnp.int32), item.astype(jnp.int32), utab, itab)
